# baseline (speedup 1.0000x reference)
import functools

import jax
import jax.numpy as jnp
from jax import lax
from jax.experimental import pallas as pl
from jax.experimental.pallas import tpu as pltpu
from jax.experimental.pallas import tpu_sc as plsc

_EMBED = 64
_NC = 2
_NS = 16
_NW = _NC * _NS
_L = 16
_W = 512


@functools.lru_cache(maxsize=None)
def _make_relayout(V: int):
    n_blocks = V // _W
    n_iters = -(-n_blocks // _NW)
    tail = V - n_blocks * _W
    pitch = _W + 1
    mesh = plsc.VectorSubcoreMesh(core_axis_name="c", subcore_axis_name="s")

    @functools.partial(
        pl.kernel,
        mesh=mesh,
        out_type=jax.ShapeDtypeStruct((V // 2, 2 * _EMBED), jnp.float32),
        scratch_types=[
            pltpu.VMEM((_EMBED, pitch), jnp.float32),
            pltpu.VMEM((_W // 2, 2 * _EMBED), jnp.float32),
            pltpu.SemaphoreType.DMA,
        ],
        compiler_params=pltpu.CompilerParams(use_tc_tiling_on_sc=True,
                                             needs_layout_passes=False),
    )
    def relayout(tT_hbm, tail_hbm, out_hbm, buf_in, buf_out, sem):
        wid = lax.axis_index("s") * _NC + lax.axis_index("c")
        rows = [jax.lax.iota(jnp.int32, _L) + _L * m for m in range(4)]

        def body(t, carry):
            g = wid + t * _NW

            @pl.when(g < n_blocks)
            def _():
                c0 = pl.multiple_of(g * _W, 128)
                o0 = pl.multiple_of(g * (_W // 2), 8)
                pltpu.sync_copy(tT_hbm.at[:, pl.ds(c0, _W)],
                                buf_in.at[:, pl.ds(0, _W)])

                def tr(r, c2):
                    for q in range(2 * _EMBED // _L):
                        col = jnp.full((_L,), 2 * r + q // 4, jnp.int32)
                        vals = plsc.load_gather(buf_in, [rows[q % 4], col])
                        buf_out[r, pl.ds(q * _L, _L)] = vals
                    return c2

                lax.fori_loop(0, _W // 2, tr, 0)
                pltpu.sync_copy(buf_out, out_hbm.at[pl.ds(o0, _W // 2)])

            return carry

        lax.fori_loop(0, n_iters, body, 0)

        @pl.when(wid == 0)
        def _():
            pltpu.sync_copy(tail_hbm, buf_out.at[pl.ds(0, tail // 2)])
            pltpu.sync_copy(buf_out.at[pl.ds(0, tail // 2)],
                            out_hbm.at[pl.ds(n_blocks * (_W // 2), tail // 2)])

    return relayout


@functools.lru_cache(maxsize=None)
def _make_gather(B: int, V: int):
    b_per_w = B // _NW
    C = 160
    n_chunks = b_per_w // C
    mesh = plsc.VectorSubcoreMesh(core_axis_name="c", subcore_axis_name="s")

    @functools.partial(
        pl.kernel,
        mesh=mesh,
        out_type=jax.ShapeDtypeStruct((B, 2 * _EMBED), jnp.float32),
        scratch_types=[
            pltpu.VMEM((C,), jnp.int32),
            pltpu.VMEM((C, _EMBED), jnp.float32),
            pltpu.SemaphoreType.DMA,
        ],
        compiler_params=pltpu.CompilerParams(use_tc_tiling_on_sc=False),
    )
    def gather(idx_hbm, table_hbm, out_hbm, idx_v, rows_v, sem):
        wid = lax.axis_index("s") * _NC + lax.axis_index("c")
        base = wid * b_per_w

        def body(j, carry):
            off = base + j * C
            pltpu.sync_copy(idx_hbm.at[pl.ds(off, C)], idx_v)
            pltpu.async_copy(table_hbm.at[idx_v], rows_v, sem).wait()
            pltpu.sync_copy(rows_v, out_hbm.at[pl.ds(off, C), pl.ds(0, _EMBED)])
            return carry

        lax.fori_loop(0, n_chunks, body, 0)

    return gather


def kernel(x, table):
    B = x.shape[0] * x.shape[1]
    V = table.shape[0]
    n_main = (V // _W) * _W
    t_tail = table[n_main:].reshape(-1, 2 * _EMBED)
    t_lin = _make_relayout(V)(table.T, t_tail)
    out128 = _make_gather(B, V)(x.reshape(B), t_lin.reshape(V, _EMBED))
    return out128[:, :_EMBED].reshape(x.shape[0], x.shape[1], _EMBED)


# speedup vs baseline: 1.6086x; 1.6086x over previous
import functools

import jax
import jax.numpy as jnp
from jax import lax
from jax.experimental import pallas as pl
from jax.experimental.pallas import tpu as pltpu
from jax.experimental.pallas import tpu_sc as plsc

_EMBED = 64
_NC = 2
_NS = 16
_NW = _NC * _NS
_L = 16
_W = 512


@functools.lru_cache(maxsize=None)
def _make_relayout(V: int):
    n_blocks = V // _W
    n_iters = -(-n_blocks // _NW)
    tail = V - n_blocks * _W
    pitch = _W + 1
    mesh = plsc.VectorSubcoreMesh(core_axis_name="c", subcore_axis_name="s")

    @functools.partial(
        pl.kernel,
        mesh=mesh,
        out_type=jax.ShapeDtypeStruct((V // 2, 2 * _EMBED), jnp.float32),
        scratch_types=[
            pltpu.VMEM((_EMBED, pitch), jnp.float32),
            pltpu.VMEM((_W // 2, 2 * _EMBED), jnp.float32),
            pltpu.SemaphoreType.DMA,
        ],
        compiler_params=pltpu.CompilerParams(use_tc_tiling_on_sc=True,
                                             needs_layout_passes=False),
    )
    def relayout(tT_hbm, tail_hbm, out_hbm, buf_in, buf_out, sem):
        wid = lax.axis_index("s") * _NC + lax.axis_index("c")
        rows = [jax.lax.iota(jnp.int32, _L) + _L * m for m in range(4)]

        def body(t, carry):
            g = wid + t * _NW

            @pl.when(g < n_blocks)
            def _():
                c0 = pl.multiple_of(g * _W, 128)
                o0 = pl.multiple_of(g * (_W // 2), 8)
                pltpu.sync_copy(tT_hbm.at[:, pl.ds(c0, _W)],
                                buf_in.at[:, pl.ds(0, _W)])

                init = (jnp.zeros((_L,), jnp.int32), jnp.ones((_L,), jnp.int32))

                @plsc.parallel_loop(0, _W // 2, unroll=8, carry=init)
                def tr(r, cols):
                    col_a, col_b = cols
                    for q in range(2 * _EMBED // _L):
                        col = col_a if q < 4 else col_b
                        vals = plsc.load_gather(buf_in, [rows[q % 4], col])
                        buf_out[r, pl.ds(q * _L, _L)] = vals
                    return (col_a + 2, col_b + 2)
                pltpu.sync_copy(buf_out, out_hbm.at[pl.ds(o0, _W // 2)])

            return carry

        lax.fori_loop(0, n_iters, body, 0)

        @pl.when(wid == 0)
        def _():
            pltpu.sync_copy(tail_hbm, buf_out.at[pl.ds(0, tail // 2)])
            pltpu.sync_copy(buf_out.at[pl.ds(0, tail // 2)],
                            out_hbm.at[pl.ds(n_blocks * (_W // 2), tail // 2)])

    return relayout


@functools.lru_cache(maxsize=None)
def _make_gather(B: int, V: int):
    b_per_w = B // _NW
    C = 640
    n_chunks = b_per_w // C
    mesh = plsc.VectorSubcoreMesh(core_axis_name="c", subcore_axis_name="s")

    @functools.partial(
        pl.kernel,
        mesh=mesh,
        out_type=jax.ShapeDtypeStruct((B, 2 * _EMBED), jnp.float32),
        scratch_types=[
            pltpu.VMEM((C,), jnp.int32),
            pltpu.VMEM((C, _EMBED), jnp.float32),
            pltpu.SemaphoreType.DMA,
        ],
        compiler_params=pltpu.CompilerParams(use_tc_tiling_on_sc=False),
    )
    def gather(idx_hbm, table_hbm, out_hbm, idx_v, rows_v, sem):
        wid = lax.axis_index("s") * _NC + lax.axis_index("c")
        base = wid * b_per_w

        def body(j, carry):
            off = base + j * C
            pltpu.sync_copy(idx_hbm.at[pl.ds(off, C)], idx_v)
            pltpu.async_copy(table_hbm.at[idx_v], rows_v, sem).wait()
            pltpu.sync_copy(rows_v, out_hbm.at[pl.ds(off, C), pl.ds(0, _EMBED)])
            return carry

        lax.fori_loop(0, n_chunks, body, 0)

    return gather


def kernel(x, table):
    B = x.shape[0] * x.shape[1]
    V = table.shape[0]
    n_main = (V // _W) * _W
    t_tail = table[n_main:].reshape(-1, 2 * _EMBED)
    t_lin = _make_relayout(V)(table.T, t_tail)
    out128 = _make_gather(B, V)(x.reshape(B), t_lin.reshape(V, _EMBED))
    return out128[:, :_EMBED].reshape(x.shape[0], x.shape[1], _EMBED)
